# baseline (speedup 1.0000x reference)
import jax
import jax.numpy as jnp
from jax import lax
from jax.experimental import pallas as pl
from jax.experimental.pallas import tpu as pltpu
from jax.experimental.pallas import tpu_sc as plsc

M, D, B = 100000, 128, 16384
NC, NS, L = 2, 16, 16
NW = NC * NS
RPW = B // NW
CH = 64
NCH = RPW // CH
NBUF = 3
CW = B // NCH
CWV = CW // L


def _body(mem_hbm, widx_hbm, wval_hbm, ridx_hbm, out_hbm,
          table_v, widx2_v, ridx_v, cpos_v, csrc_v, rbuf,
          sg0, sg1, sg2, so0, so1, so2, sw0, sw1, se_g, se_g2, se_s, se_s2):
  wid = lax.axis_index("s") * NC + lax.axis_index("c")
  rbase = wid * RPW
  iota = lax.iota(jnp.int32, L)
  sgs = (sg0, sg1, sg2)
  sos = (so0, so1, so2)
  sws = (sw0, sw1)

  pltpu.sync_copy(ridx_hbm.at[pl.ds(rbase, RPW)], ridx_v)
  g_desc = {}
  o_desc = {}
  w_desc = {}
  for k in range(NBUF):
    g_desc[k] = pltpu.async_copy(
        mem_hbm.at[ridx_v.at[pl.ds(k * CH, CH)]], rbuf.at[k], sgs[k])
  w_desc[0] = pltpu.async_copy(widx_hbm.at[pl.ds(0, CW)], widx2_v.at[0], sws[0])

  @plsc.parallel_loop(0, RPW // L, unroll=4)
  def _(i):
    rv = ridx_v[pl.ds(i * L, L)]
    plsc.store_scatter(table_v, [rv], jnp.zeros_like(iota))

  for k in range(NCH):
    slot = k % NBUF
    par = k % 2
    g_desc[k].wait()
    o_desc[k] = pltpu.async_copy(
        rbuf.at[slot], out_hbm.at[pl.ds(rbase + k * CH, CH)], sos[slot])

    w_desc[k].wait()
    if k + 1 < NCH:
      w_desc[k + 1] = pltpu.async_copy(
          widx_hbm.at[pl.ds((k + 1) * CW, CW)], widx2_v.at[(k + 1) % 2],
          sws[(k + 1) % 2])

    @plsc.parallel_loop(0, CWV, unroll=8)
    def _(i, k=k, par=par):
      idxv = widx2_v[par, pl.ds(i * L, L)]
      jv = (1 + k * CW) + i * L + iota
      plsc.store_scatter(table_v, [idxv], jv)

    def chk_body(i, a, k=k, par=par):
      idxv = widx2_v[par, pl.ds(i * L, L)]
      jv = (1 + k * CW) + i * L + iota
      g = plsc.load_gather(table_v, [idxv])
      return a | (g < jv)

    acc = lax.fori_loop(0, CWV, chk_body, iota < 0, unroll=8)

    @pl.when(jnp.any(acc))
    def _(k=k, par=par):
      def fix_body(i, _):
        idxv = widx2_v[par, pl.ds(i * L, L)]
        _, lastm = plsc.scan_count(idxv)
        jv = (1 + k * CW) + i * L + iota
        g = plsc.load_gather(table_v, [idxv])
        plsc.store_scatter(table_v, [idxv], jv, mask=lastm & (g < jv))
        return 0

      lax.fori_loop(0, CWV, fix_body, 0)

    nk = k + NBUF
    if nk < NCH:
      o_desc[k].wait()
      g_desc[nk] = pltpu.async_copy(
          mem_hbm.at[ridx_v.at[pl.ds(nk * CH, CH)]], rbuf.at[slot], sgs[slot])
  for k in range(NCH - NBUF, NCH):
    o_desc[k].wait()

  def d_body(i, nv):
    rv = ridx_v[pl.ds(i * L, L)]
    tv = plsc.load_gather(table_v, [rv])
    hit = tv > 0
    pos = rbase + i * L + iota
    plsc.store_compressed(cpos_v.at[pl.ds(nv, L)], pos, mask=hit)
    plsc.store_compressed(csrc_v.at[pl.ds(nv, L)], tv - 1, mask=hit)
    return nv + jnp.sum(hit.astype(jnp.int32))

  nv = lax.fori_loop(0, RPW // L, d_body, 0)

  safe = jnp.maximum(nv - 1, 0)
  fill_p = jnp.full((L,), cpos_v[pl.ds(safe, L)][0], jnp.int32)
  fill_s = jnp.full((L,), csrc_v[pl.ds(safe, L)][0], jnp.int32)
  cpos_v[pl.ds(nv, L)] = fill_p
  csrc_v[pl.ds(nv, L)] = fill_s
  cpos_v[pl.ds(nv + L, L)] = fill_p
  csrc_v[pl.ds(nv + L, L)] = fill_s

  nE = (nv + L - 1) // L
  eslot0 = rbuf.at[0, pl.ds(0, L)]
  eslot1 = rbuf.at[0, pl.ds(L, L)]

  def e_pair(m, _):
    k0 = 2 * m
    k1 = k0 + 1
    sv0 = csrc_v[pl.ds(k0 * L, L)]
    pv0 = cpos_v[pl.ds(k0 * L, L)]
    g0 = pltpu.async_copy(wval_hbm.at[sv0], eslot0, se_g)
    run1 = k1 < nE

    @pl.when(run1)
    def _():
      sv1 = csrc_v[pl.ds(k1 * L, L)]
      pltpu.async_copy(wval_hbm.at[sv1], eslot1, se_g2)

    g0.wait()
    s0 = pltpu.async_copy(eslot0, out_hbm.at[pv0], se_s)

    @pl.when(run1)
    def _():
      sv1 = csrc_v[pl.ds(k1 * L, L)]
      pv1 = cpos_v[pl.ds(k1 * L, L)]
      pltpu.make_async_copy(wval_hbm.at[sv1], eslot1, se_g2).wait()
      pltpu.async_copy(eslot1, out_hbm.at[pv1], se_s2).wait()

    s0.wait()
    return 0

  lax.fori_loop(0, (nE + 1) // 2, e_pair, 0)


_mrw = pl.kernel(
    _body,
    out_type=jax.ShapeDtypeStruct((B, D), jnp.float32),
    mesh=plsc.VectorSubcoreMesh(core_axis_name="c", subcore_axis_name="s",
                                num_cores=NC, num_subcores=NS),
    compiler_params=pltpu.CompilerParams(needs_layout_passes=False, skip_device_barrier=True, disable_bounds_checks=True, disable_semaphore_checks=True),
    scratch_types=[
        pltpu.VMEM((M,), jnp.int32),
        pltpu.VMEM((2, CW), jnp.int32),
        pltpu.VMEM((RPW,), jnp.int32),
        pltpu.VMEM((RPW + 2 * L,), jnp.int32),
        pltpu.VMEM((RPW + 2 * L,), jnp.int32),
        pltpu.VMEM((NBUF, CH, D), jnp.float32),
        pltpu.SemaphoreType.DMA,
        pltpu.SemaphoreType.DMA,
        pltpu.SemaphoreType.DMA,
        pltpu.SemaphoreType.DMA,
        pltpu.SemaphoreType.DMA,
        pltpu.SemaphoreType.DMA,
        pltpu.SemaphoreType.DMA,
        pltpu.SemaphoreType.DMA,
        pltpu.SemaphoreType.DMA,
        pltpu.SemaphoreType.DMA,
        pltpu.SemaphoreType.DMA,
        pltpu.SemaphoreType.DMA,
    ],
)


def kernel(mem, write_idx, write_val, read_idx):
  return _mrw(mem, write_idx, write_val, read_idx)


# speedup vs baseline: 1.1344x; 1.1344x over previous
import jax
import jax.numpy as jnp
from jax import lax
from jax.experimental import pallas as pl
from jax.experimental.pallas import tpu as pltpu
from jax.experimental.pallas import tpu_sc as plsc

M, D, B = 100000, 128, 16384
NC, NS, L = 2, 16, 16
NW = NC * NS
RPW = B // NW
CH = 64
NCH = RPW // CH
NBUF = 3
CW = B // NCH
CWV = CW // L


def _body(mem_hbm, widx_hbm, wval_hbm, ridx_hbm, out_hbm,
          table_v, widx2_v, ridx_v, cpos_v, csrc_v, rbuf,
          sg0, sg1, sg2, so0, so1, so2, sw0, sw1, se_g, se_g2, se_s, se_s2):
  wid = lax.axis_index("s") * NC + lax.axis_index("c")
  rbase = wid * RPW
  iota = lax.iota(jnp.int32, L)
  sgs = (sg0, sg1, sg2)
  sos = (so0, so1, so2)
  sws = (sw0, sw1)

  pltpu.sync_copy(ridx_hbm.at[pl.ds(rbase, RPW)], ridx_v)
  g_desc = {}
  o_desc = {}
  w_desc = {}
  for k in range(NBUF):
    g_desc[k] = pltpu.async_copy(
        mem_hbm.at[ridx_v.at[pl.ds(k * CH, CH)]], rbuf.at[k], sgs[k])
  w_desc[0] = pltpu.async_copy(widx_hbm.at[pl.ds(0, CW)], widx2_v.at[0], sws[0])

  @plsc.parallel_loop(0, RPW // L, unroll=4)
  def _(i):
    rv = ridx_v[pl.ds(i * L, L)]
    plsc.store_scatter(table_v, [rv], jnp.zeros_like(iota))

  for k in range(NCH):
    slot = k % NBUF
    par = k % 2
    g_desc[k].wait()
    o_desc[k] = pltpu.async_copy(
        rbuf.at[slot], out_hbm.at[pl.ds(rbase + k * CH, CH)], sos[slot])

    w_desc[k].wait()
    if k + 1 < NCH:
      w_desc[k + 1] = pltpu.async_copy(
          widx_hbm.at[pl.ds((k + 1) * CW, CW)], widx2_v.at[(k + 1) % 2],
          sws[(k + 1) % 2])

    @plsc.parallel_loop(0, CWV, unroll=2)
    def _(i, k=k, par=par):
      idxv = widx2_v[par, pl.ds(i * L, L)]
      jv = (1 + k * CW) + i * L + iota
      plsc.store_scatter(table_v, [idxv], jv)

    def chk_body(i, a, k=k, par=par):
      idxv = widx2_v[par, pl.ds(i * L, L)]
      jv = (1 + k * CW) + i * L + iota
      g = plsc.load_gather(table_v, [idxv])
      return a | (g < jv)

    acc = lax.fori_loop(0, CWV, chk_body, iota < 0, unroll=8)

    @pl.when(jnp.any(acc))
    def _(k=k, par=par):
      def fix_body(i, _):
        idxv = widx2_v[par, pl.ds(i * L, L)]
        _, lastm = plsc.scan_count(idxv)
        jv = (1 + k * CW) + i * L + iota
        g = plsc.load_gather(table_v, [idxv])
        plsc.store_scatter(table_v, [idxv], jv, mask=lastm & (g < jv))
        return 0

      lax.fori_loop(0, CWV, fix_body, 0)

    nk = k + NBUF
    if nk < NCH:
      o_desc[k].wait()
      g_desc[nk] = pltpu.async_copy(
          mem_hbm.at[ridx_v.at[pl.ds(nk * CH, CH)]], rbuf.at[slot], sgs[slot])
  for k in range(NCH - NBUF, NCH):
    o_desc[k].wait()

  def d_body(i, nv):
    rv = ridx_v[pl.ds(i * L, L)]
    tv = plsc.load_gather(table_v, [rv])
    hit = tv > 0
    pos = rbase + i * L + iota
    plsc.store_compressed(cpos_v.at[pl.ds(nv, L)], pos, mask=hit)
    plsc.store_compressed(csrc_v.at[pl.ds(nv, L)], tv - 1, mask=hit)
    return nv + jnp.sum(hit.astype(jnp.int32))

  nv = lax.fori_loop(0, RPW // L, d_body, 0)

  safe = jnp.maximum(nv - 1, 0)
  fill_p = jnp.full((L,), cpos_v[pl.ds(safe, L)][0], jnp.int32)
  fill_s = jnp.full((L,), csrc_v[pl.ds(safe, L)][0], jnp.int32)
  cpos_v[pl.ds(nv, L)] = fill_p
  csrc_v[pl.ds(nv, L)] = fill_s
  cpos_v[pl.ds(nv + L, L)] = fill_p
  csrc_v[pl.ds(nv + L, L)] = fill_s

  nE = (nv + L - 1) // L
  eslot0 = rbuf.at[0, pl.ds(0, L)]
  eslot1 = rbuf.at[0, pl.ds(L, L)]

  def e_pair(m, _):
    k0 = 2 * m
    k1 = k0 + 1
    sv0 = csrc_v[pl.ds(k0 * L, L)]
    pv0 = cpos_v[pl.ds(k0 * L, L)]
    g0 = pltpu.async_copy(wval_hbm.at[sv0], eslot0, se_g)
    run1 = k1 < nE

    @pl.when(run1)
    def _():
      sv1 = csrc_v[pl.ds(k1 * L, L)]
      pltpu.async_copy(wval_hbm.at[sv1], eslot1, se_g2)

    g0.wait()
    s0 = pltpu.async_copy(eslot0, out_hbm.at[pv0], se_s)

    @pl.when(run1)
    def _():
      sv1 = csrc_v[pl.ds(k1 * L, L)]
      pv1 = cpos_v[pl.ds(k1 * L, L)]
      pltpu.make_async_copy(wval_hbm.at[sv1], eslot1, se_g2).wait()
      pltpu.async_copy(eslot1, out_hbm.at[pv1], se_s2).wait()

    s0.wait()
    return 0

  lax.fori_loop(0, (nE + 1) // 2, e_pair, 0)


_mrw = pl.kernel(
    _body,
    out_type=jax.ShapeDtypeStruct((B, D), jnp.float32),
    mesh=plsc.VectorSubcoreMesh(core_axis_name="c", subcore_axis_name="s",
                                num_cores=NC, num_subcores=NS),
    compiler_params=pltpu.CompilerParams(needs_layout_passes=False),
    scratch_types=[
        pltpu.VMEM((M,), jnp.int32),
        pltpu.VMEM((2, CW), jnp.int32),
        pltpu.VMEM((RPW,), jnp.int32),
        pltpu.VMEM((RPW + 2 * L,), jnp.int32),
        pltpu.VMEM((RPW + 2 * L,), jnp.int32),
        pltpu.VMEM((NBUF, CH, D), jnp.float32),
        pltpu.SemaphoreType.DMA,
        pltpu.SemaphoreType.DMA,
        pltpu.SemaphoreType.DMA,
        pltpu.SemaphoreType.DMA,
        pltpu.SemaphoreType.DMA,
        pltpu.SemaphoreType.DMA,
        pltpu.SemaphoreType.DMA,
        pltpu.SemaphoreType.DMA,
        pltpu.SemaphoreType.DMA,
        pltpu.SemaphoreType.DMA,
        pltpu.SemaphoreType.DMA,
        pltpu.SemaphoreType.DMA,
    ],
)


def kernel(mem, write_idx, write_val, read_idx):
  return _mrw(mem, write_idx, write_val, read_idx)


# speedup vs baseline: 1.1448x; 1.0091x over previous
import jax
import jax.numpy as jnp
from jax import lax
from jax.experimental import pallas as pl
from jax.experimental.pallas import tpu as pltpu
from jax.experimental.pallas import tpu_sc as plsc

M, D, B = 100000, 128, 16384
NC, NS, L = 2, 16, 16
NW = NC * NS
RPW = B // NW
CH = 64
NCH = RPW // CH
NBUF = 3
CW = B // NCH
CWV = CW // L


def _body(mem_hbm, widx_hbm, wval_hbm, ridx_hbm, out_hbm,
          table_v, widx2_v, ridx_v, cpos_v, csrc_v, rbuf,
          sg0, sg1, sg2, so0, so1, so2, sw0, sw1, se_g, se_g2, se_s, se_s2):
  wid = lax.axis_index("s") * NC + lax.axis_index("c")
  rbase = wid * RPW
  iota = lax.iota(jnp.int32, L)
  sgs = (sg0, sg1, sg2)
  sos = (so0, so1, so2)
  sws = (sw0, sw1)

  pltpu.sync_copy(ridx_hbm.at[pl.ds(rbase, RPW)], ridx_v)
  g_desc = {}
  o_desc = {}
  w_desc = {}
  for k in range(NBUF):
    g_desc[k] = pltpu.async_copy(
        mem_hbm.at[ridx_v.at[pl.ds(k * CH, CH)]], rbuf.at[k], sgs[k])
  w_desc[0] = pltpu.async_copy(widx_hbm.at[pl.ds(0, CW)], widx2_v.at[0], sws[0])

  @plsc.parallel_loop(0, RPW // L, unroll=4)
  def _(i):
    rv = ridx_v[pl.ds(i * L, L)]
    plsc.store_scatter(table_v, [rv], jnp.zeros_like(iota))

  for k in range(NCH):
    slot = k % NBUF
    par = k % 2
    g_desc[k].wait()
    o_desc[k] = pltpu.async_copy(
        rbuf.at[slot], out_hbm.at[pl.ds(rbase + k * CH, CH)], sos[slot])

    w_desc[k].wait()
    if k + 1 < NCH:
      w_desc[k + 1] = pltpu.async_copy(
          widx_hbm.at[pl.ds((k + 1) * CW, CW)], widx2_v.at[(k + 1) % 2],
          sws[(k + 1) % 2])

    @plsc.parallel_loop(0, CWV, unroll=4)
    def _(i, k=k, par=par):
      idxv = widx2_v[par, pl.ds(i * L, L)]
      jv = (1 + k * CW) + i * L + iota
      plsc.store_scatter(table_v, [idxv], jv)

    def chk_body(i, a, k=k, par=par):
      idxv = widx2_v[par, pl.ds(i * L, L)]
      jv = (1 + k * CW) + i * L + iota
      g = plsc.load_gather(table_v, [idxv])
      return a | (g < jv)

    acc = lax.fori_loop(0, CWV, chk_body, iota < 0, unroll=8)

    @pl.when(jnp.any(acc))
    def _(k=k, par=par):
      def fix_body(i, _):
        idxv = widx2_v[par, pl.ds(i * L, L)]
        _, lastm = plsc.scan_count(idxv)
        jv = (1 + k * CW) + i * L + iota
        g = plsc.load_gather(table_v, [idxv])
        plsc.store_scatter(table_v, [idxv], jv, mask=lastm & (g < jv))
        return 0

      lax.fori_loop(0, CWV, fix_body, 0)

    nk = k + NBUF
    if nk < NCH:
      o_desc[k].wait()
      g_desc[nk] = pltpu.async_copy(
          mem_hbm.at[ridx_v.at[pl.ds(nk * CH, CH)]], rbuf.at[slot], sgs[slot])
  for k in range(NCH - NBUF, NCH):
    o_desc[k].wait()

  def d_body(i, nv):
    rv = ridx_v[pl.ds(i * L, L)]
    tv = plsc.load_gather(table_v, [rv])
    hit = tv > 0
    pos = rbase + i * L + iota
    plsc.store_compressed(cpos_v.at[pl.ds(nv, L)], pos, mask=hit)
    plsc.store_compressed(csrc_v.at[pl.ds(nv, L)], tv - 1, mask=hit)
    return nv + jnp.sum(hit.astype(jnp.int32))

  nv = lax.fori_loop(0, RPW // L, d_body, 0)

  safe = jnp.maximum(nv - 1, 0)
  fill_p = jnp.full((L,), cpos_v[pl.ds(safe, L)][0], jnp.int32)
  fill_s = jnp.full((L,), csrc_v[pl.ds(safe, L)][0], jnp.int32)
  cpos_v[pl.ds(nv, L)] = fill_p
  csrc_v[pl.ds(nv, L)] = fill_s
  cpos_v[pl.ds(nv + L, L)] = fill_p
  csrc_v[pl.ds(nv + L, L)] = fill_s

  nE = (nv + L - 1) // L
  eslot0 = rbuf.at[0, pl.ds(0, L)]
  eslot1 = rbuf.at[0, pl.ds(L, L)]

  def e_pair(m, _):
    k0 = 2 * m
    k1 = k0 + 1
    sv0 = csrc_v[pl.ds(k0 * L, L)]
    pv0 = cpos_v[pl.ds(k0 * L, L)]
    g0 = pltpu.async_copy(wval_hbm.at[sv0], eslot0, se_g)
    run1 = k1 < nE

    @pl.when(run1)
    def _():
      sv1 = csrc_v[pl.ds(k1 * L, L)]
      pltpu.async_copy(wval_hbm.at[sv1], eslot1, se_g2)

    g0.wait()
    s0 = pltpu.async_copy(eslot0, out_hbm.at[pv0], se_s)

    @pl.when(run1)
    def _():
      sv1 = csrc_v[pl.ds(k1 * L, L)]
      pv1 = cpos_v[pl.ds(k1 * L, L)]
      pltpu.make_async_copy(wval_hbm.at[sv1], eslot1, se_g2).wait()
      pltpu.async_copy(eslot1, out_hbm.at[pv1], se_s2).wait()

    s0.wait()
    return 0

  lax.fori_loop(0, (nE + 1) // 2, e_pair, 0)


_mrw = pl.kernel(
    _body,
    out_type=jax.ShapeDtypeStruct((B, D), jnp.float32),
    mesh=plsc.VectorSubcoreMesh(core_axis_name="c", subcore_axis_name="s",
                                num_cores=NC, num_subcores=NS),
    compiler_params=pltpu.CompilerParams(needs_layout_passes=False),
    scratch_types=[
        pltpu.VMEM((M,), jnp.int32),
        pltpu.VMEM((2, CW), jnp.int32),
        pltpu.VMEM((RPW,), jnp.int32),
        pltpu.VMEM((RPW + 2 * L,), jnp.int32),
        pltpu.VMEM((RPW + 2 * L,), jnp.int32),
        pltpu.VMEM((NBUF, CH, D), jnp.float32),
        pltpu.SemaphoreType.DMA,
        pltpu.SemaphoreType.DMA,
        pltpu.SemaphoreType.DMA,
        pltpu.SemaphoreType.DMA,
        pltpu.SemaphoreType.DMA,
        pltpu.SemaphoreType.DMA,
        pltpu.SemaphoreType.DMA,
        pltpu.SemaphoreType.DMA,
        pltpu.SemaphoreType.DMA,
        pltpu.SemaphoreType.DMA,
        pltpu.SemaphoreType.DMA,
        pltpu.SemaphoreType.DMA,
    ],
)


def kernel(mem, write_idx, write_val, read_idx):
  return _mrw(mem, write_idx, write_val, read_idx)
